# baseline (speedup 1.0000x reference)
import functools

import jax
import jax.numpy as jnp
from jax import lax
from jax.experimental import pallas as pl
from jax.experimental.pallas import tpu as pltpu
from jax.experimental.pallas import tpu_sc as plsc

_N = 10000
_E = 640000
_G = 64
_C = 128
_NC = 2
_NS = 16
_PT = 320
_EPAD = _NS * _PT * _C
_CHUNKS = _EPAD // _C
_BLK = 20
_NB = _PT // _BLK
_WTILES = 10
_RPS = _N // _WTILES

_mesh = plsc.VectorSubcoreMesh(
    core_axis_name="c", subcore_axis_name="s",
    num_cores=_NC, num_subcores=_NS)


def _make_agg(F, NPASS):
    @functools.partial(
        pl.kernel,
        out_type=jax.ShapeDtypeStruct((NPASS, _NC, _N, F), jnp.float32),
        mesh=_mesh,
        scratch_types=[
            pltpu.VMEM((_BLK, _C), jnp.int32),
            pltpu.VMEM((_BLK, _C), jnp.int32),
            pltpu.VMEM((_BLK, _C), jnp.float32),
            pltpu.VMEM((_C, F), jnp.float32),
            pltpu.VMEM((_C, F), jnp.float32),
            pltpu.VMEM((_C, F), jnp.float32),
            pltpu.VMEM((_C, F), jnp.float32),
            pltpu.VMEM_SHARED((_N, F), jnp.float32),
            pltpu.SemaphoreType.DMA,
            pltpu.SemaphoreType.DMA,
            pltpu.SemaphoreType.DMA,
            pltpu.SemaphoreType.DMA,
        ],
        compiler_params=pltpu.CompilerParams(use_tc_tiling_on_sc=False),
    )
    def agg(tab_hbm, src_hbm, dst_hbm, w_hbm, out_hbm,
            src_v, dst_v, w_v, g0, g1, sc0, sc1, acc,
            gs0, gs1, ss0, ss1):
        cid = lax.axis_index("c")
        sid = lax.axis_index("s")
        base = sid * _RPS
        q_full = _RPS // _C
        rem = _RPS - q_full * _C
        GB, SB = (g0, g1), (sc0, sc1)
        GS, SS = (gs0, gs1), (ss0, ss1)

        def issue_gather(i, par):
            pltpu.async_copy(tab_hbm.at[src_v.at[i]], GB[par], GS[par])

        def wait_gather(par):
            pltpu.make_async_copy(tab_hbm.at[src_v.at[0]],
                                  GB[par], GS[par]).wait()

        def issue_scatter(i, par):
            pltpu.async_copy(SB[par], acc.at[dst_v.at[i]], SS[par], add=True)

        def wait_scatter(par):
            pltpu.make_async_copy(SB[par], acc.at[pl.ds(0, _C)],
                                  SS[par]).wait()

        def scale(i, par):
            gbuf, sbuf = GB[par], SB[par]

            def rowblk(rb, _):
                r0 = rb * 16
                wvec = w_v[i, pl.ds(r0, 16)]
                for t in range(16):
                    wv = wvec[t]
                    for k in range(F // 16):
                        sl = pl.ds(k * 16, 16)
                        sbuf[r0 + t, sl] = gbuf[r0 + t, sl] * wv
                return 0
            lax.fori_loop(0, _C // 16, rowblk, 0)

        for p in range(NPASS):
            def zr(r, _):
                for k in range(F // 16):
                    sc0[r, pl.ds(k * 16, 16)] = jnp.zeros((16,), jnp.float32)
                return 0
            lax.fori_loop(0, _C, zr, 0)

            @pl.when(sid < _WTILES)
            def _():
                for q in range(q_full):
                    pltpu.sync_copy(sc0, acc.at[pl.ds(base + q * _C, _C)])
                pltpu.sync_copy(sc0.at[pl.ds(0, rem)],
                                acc.at[pl.ds(base + q_full * _C, rem)])
            plsc.subcore_barrier()

            def blk(bi, _):
                b0 = bi * _BLK
                pltpu.sync_copy(src_hbm.at[p, cid, sid, pl.ds(b0, _BLK)], src_v)
                pltpu.sync_copy(dst_hbm.at[cid, sid, pl.ds(b0, _BLK)], dst_v)
                pltpu.sync_copy(w_hbm.at[cid, sid, pl.ds(b0, _BLK)], w_v)

                issue_gather(0, 0)
                issue_gather(1, 1)
                for par in (0, 1):
                    wait_gather(par)
                    scale(par, par)
                    issue_scatter(par, par)
                    issue_gather(par + 2, par)

                def pair(kk, _):
                    for par in (0, 1):
                        i = kk * 2 + par
                        wait_gather(par)
                        wait_scatter(par)
                        scale(i, par)
                        issue_scatter(i, par)
                        issue_gather(i + 2, par)
                    return 0
                lax.fori_loop(1, _BLK // 2 - 1, pair, 0)

                for par in (0, 1):
                    i = _BLK - 2 + par
                    wait_gather(par)
                    wait_scatter(par)
                    scale(i, par)
                    issue_scatter(i, par)
                wait_scatter(0)
                wait_scatter(1)
                return 0
            lax.fori_loop(0, _NB, blk, 0)
            plsc.subcore_barrier()

            @pl.when(sid < _WTILES)
            def _():
                pltpu.sync_copy(acc.at[pl.ds(base, _RPS)],
                                out_hbm.at[p, cid, pl.ds(base, _RPS)])
            plsc.subcore_barrier()

    return agg


_agg_16 = _make_agg(16, 1)
_agg_32 = _make_agg(32, 1)
_agg_64_1 = _make_agg(64, 1)
_agg_64_2 = _make_agg(64, 2)


def _stage0_body(degp_ref, x_ref, w1_ref, dinv_ref, h1_ref, hs1_ref):
    deg = degp_ref[:, 0:1] + 1.0
    dinv = lax.rsqrt(jnp.maximum(deg, 1e-12))
    dinv_ref[...] = dinv
    h1 = jnp.dot(x_ref[...], w1_ref[...], preferred_element_type=jnp.float32)
    h1_ref[...] = h1
    hs1_ref[...] = h1 * dinv


def _stage0(degp, x, w1):
    n, fo = x.shape[0], w1.shape[1]
    return pl.pallas_call(
        _stage0_body,
        out_shape=(
            jax.ShapeDtypeStruct((n, 1), jnp.float32),
            jax.ShapeDtypeStruct((n, fo), jnp.float32),
            jax.ShapeDtypeStruct((n, fo), jnp.float32),
        ),
    )(degp, x, w1)


def _combine(agg, h, dinv, b, g, be):
    out = dinv * agg + (dinv * dinv) * h + b
    m = jnp.mean(out, axis=0, keepdims=True)
    v = jnp.mean((out - m) ** 2, axis=0, keepdims=True)
    return jax.nn.relu((out - m) * lax.rsqrt(v + 1e-5) * g + be)


def _mid_body(agg_ref, h_ref, dinv_ref, b_ref, g_ref, be_ref, wn_ref,
              hn_ref, hsn_ref):
    dinv = dinv_ref[...]
    y = _combine(agg_ref[...], h_ref[...], dinv,
                 b_ref[...], g_ref[...], be_ref[...])
    hn = jnp.dot(y, wn_ref[...], preferred_element_type=jnp.float32)
    hn_ref[...] = hn
    hsn_ref[...] = hn * dinv


def _mid(agg, h, dinv, b, g, be, wn):
    n, fo = h.shape[0], wn.shape[1]
    return pl.pallas_call(
        _mid_body,
        out_shape=(
            jax.ShapeDtypeStruct((n, fo), jnp.float32),
            jax.ShapeDtypeStruct((n, fo), jnp.float32),
        ),
    )(agg, h, dinv, b, g, be, wn)


def _final_body(agg_ref, h_ref, dinv_ref, b_ref, g_ref, be_ref, batch_ref,
                wp1_ref, bp1_ref, wp2_ref, bp2_ref,
                pool_ref, z_ref, x3_ref):
    x3 = _combine(agg_ref[...], h_ref[...], dinv_ref[...],
                  b_ref[...], g_ref[...], be_ref[...])
    x3_ref[...] = x3
    gid = lax.broadcasted_iota(jnp.int32, (x3.shape[0], _G), 1)
    onehot = (batch_ref[...] == gid).astype(jnp.float32)
    s = lax.dot_general(onehot, x3, (((0,), (0,)), ((), ())),
                        preferred_element_type=jnp.float32)
    c = jnp.sum(onehot, axis=0)
    pool = s / jnp.maximum(c, 1.0)[:, None]
    pool_ref[...] = pool
    z1 = jax.nn.relu(
        jnp.dot(pool, wp1_ref[...], preferred_element_type=jnp.float32)
        + bp1_ref[...])
    z_ref[...] = (jnp.dot(z1, wp2_ref[...], preferred_element_type=jnp.float32)
                  + bp2_ref[...])


def _final(agg, h, dinv, b, g, be, batch2, wp1, bp1, wp2, bp2):
    n, f = h.shape
    return pl.pallas_call(
        _final_body,
        out_shape=(
            jax.ShapeDtypeStruct((_G, f), jnp.float32),
            jax.ShapeDtypeStruct((_G, f), jnp.float32),
            jax.ShapeDtypeStruct((n, f), jnp.float32),
        ),
    )(agg, h, dinv, b, g, be, batch2, wp1, bp1, wp2, bp2)


def _r1(a):
    return a.reshape(1, -1)


def kernel(x_t, edge_index_t, edge_attr_t, batch_t,
           x_f, edge_index_f, edge_attr_f, batch_f, params):
    p = params

    def eshape(a):
        return jnp.pad(a, (0, _EPAD - _E)).reshape(_NS, _PT, _C)

    src_all = jnp.stack([eshape(edge_index_t[0]),
                         eshape(edge_index_f[0]) + _N])
    dst_all = jnp.stack([eshape(edge_index_t[1]), eshape(edge_index_f[1])])
    w_all = jnp.stack([eshape(edge_attr_t), eshape(edge_attr_f)])
    src_1 = src_all[None]
    src_2 = jnp.stack([src_all, src_all + 2 * _N])

    ones_tab = jnp.ones((2 * _N, 16), jnp.float32)
    degp = _agg_16(ones_tab, src_1, dst_all, w_all)[0]

    dinv_t, h1_t, hs1_t = _stage0(degp[0], x_t, p['W1t'])
    dinv_f, h1_f, hs1_f = _stage0(degp[1], x_f, p['W1f'])

    stacked1 = jnp.concatenate([hs1_t, hs1_f])
    tab1 = stacked1.reshape(2 * _N, 2, 64).transpose(1, 0, 2).reshape(
        4 * _N, 64)
    agg1 = _agg_64_2(tab1, src_2, dst_all, w_all)
    agg1_t = jnp.concatenate([agg1[0, 0], agg1[1, 0]], axis=1)
    agg1_f = jnp.concatenate([agg1[0, 1], agg1[1, 1]], axis=1)

    h2_t, hs2_t = _mid(agg1_t, h1_t, dinv_t, _r1(p['b1t']),
                       _r1(p['g_bn1t']), _r1(p['be_bn1t']), p['W2t'])
    h2_f, hs2_f = _mid(agg1_f, h1_f, dinv_f, _r1(p['b1f']),
                       _r1(p['g_bn1f']), _r1(p['be_bn1f']), p['W2f'])

    tab2 = jnp.concatenate([hs2_t, hs2_f])
    agg2 = _agg_64_1(tab2, src_1, dst_all, w_all)[0]
    h3_t, hs3_t = _mid(agg2[0], h2_t, dinv_t, _r1(p['b2t']),
                       _r1(p['g_bn2t']), _r1(p['be_bn2t']), p['W3t'])
    h3_f, hs3_f = _mid(agg2[1], h2_f, dinv_f, _r1(p['b2f']),
                       _r1(p['g_bn2f']), _r1(p['be_bn2f']), p['W3f'])

    tab3 = jnp.concatenate([hs3_t, hs3_f])
    agg3 = _agg_32(tab3, src_1, dst_all, w_all)[0]

    h_time, z_time, xt = _final(
        agg3[0], h3_t, dinv_t, _r1(p['b3t']), _r1(p['g_bn3t']),
        _r1(p['be_bn3t']), batch_t.reshape(_N, 1),
        p['Wp1t'], _r1(p['bp1t']), p['Wp2t'], _r1(p['bp2t']))
    h_freq, z_freq, xf = _final(
        agg3[1], h3_f, dinv_f, _r1(p['b3f']), _r1(p['g_bn3f']),
        _r1(p['be_bn3f']), batch_f.reshape(_N, 1),
        p['Wp1f'], _r1(p['bp1f']), p['Wp2f'], _r1(p['bp2f']))

    return (h_time, z_time, h_freq, z_freq, xt, xf)


# speedup vs baseline: 1.3929x; 1.3929x over previous
import functools

import jax
import jax.numpy as jnp
from jax import lax
from jax.experimental import pallas as pl
from jax.experimental.pallas import tpu as pltpu
from jax.experimental.pallas import tpu_sc as plsc

_N = 10000
_E = 640000
_G = 64
_C = 96
_NC = 2
_NS = 16
_PT = 420
_EPAD = _NS * _PT * _C
_CHUNKS = _EPAD // _C
_BLK = 42
_NB = _PT // _BLK
_WTILES = 10
_RPS = _N // _WTILES

_mesh = plsc.VectorSubcoreMesh(
    core_axis_name="c", subcore_axis_name="s",
    num_cores=_NC, num_subcores=_NS)


def _make_agg(F, NPASS):
    @functools.partial(
        pl.kernel,
        out_type=jax.ShapeDtypeStruct((NPASS, _NC, _N, F), jnp.float32),
        mesh=_mesh,
        scratch_types=[
            pltpu.VMEM((_BLK, _C), jnp.int32),
            pltpu.VMEM((_BLK, _C), jnp.int32),
            pltpu.VMEM((_BLK, _C), jnp.float32),
            pltpu.VMEM((_C, F), jnp.float32),
            pltpu.VMEM((_C, F), jnp.float32),
            pltpu.VMEM((_C, F), jnp.float32),
            pltpu.VMEM((_C, F), jnp.float32),
            pltpu.VMEM_SHARED((_N, F), jnp.float32),
            pltpu.SemaphoreType.DMA,
            pltpu.SemaphoreType.DMA,
            pltpu.SemaphoreType.DMA,
            pltpu.SemaphoreType.DMA,
        ],
        compiler_params=pltpu.CompilerParams(use_tc_tiling_on_sc=False),
    )
    def agg(tab_hbm, src_hbm, dst_hbm, w_hbm, out_hbm,
            src_v, dst_v, w_v, g0, g1, sc0, sc1, acc,
            gs0, gs1, ss0, ss1):
        cid = lax.axis_index("c")
        sid = lax.axis_index("s")
        base = sid * _RPS
        q_full = _RPS // _C
        rem = _RPS - q_full * _C
        GB, SB = (g0, g1), (sc0, sc1)
        GS, SS = (gs0, gs1), (ss0, ss1)

        def issue_gather(i, par):
            pltpu.async_copy(tab_hbm.at[src_v.at[i]], GB[par], GS[par])

        def wait_gather(par):
            pltpu.make_async_copy(tab_hbm.at[src_v.at[0]],
                                  GB[par], GS[par]).wait()

        def issue_scatter(i, par):
            pltpu.async_copy(SB[par], acc.at[dst_v.at[i]], SS[par], add=True)

        def wait_scatter(par):
            pltpu.make_async_copy(SB[par], acc.at[pl.ds(0, _C)],
                                  SS[par]).wait()

        def scale(i, par):
            gbuf, sbuf = GB[par], SB[par]

            def rowblk(rb, _):
                r0 = rb * 16
                wvec = w_v[i, pl.ds(r0, 16)]
                for t in range(16):
                    wv = wvec[t]
                    for k in range(F // 16):
                        sl = pl.ds(k * 16, 16)
                        sbuf[r0 + t, sl] = gbuf[r0 + t, sl] * wv
                return 0
            lax.fori_loop(0, _C // 16, rowblk, 0)

        for p in range(NPASS):
            def zr(r, _):
                for k in range(F // 16):
                    sc0[r, pl.ds(k * 16, 16)] = jnp.zeros((16,), jnp.float32)
                return 0
            lax.fori_loop(0, _C, zr, 0)

            @pl.when(sid < _WTILES)
            def _():
                for q in range(q_full):
                    pltpu.sync_copy(sc0, acc.at[pl.ds(base + q * _C, _C)])
                pltpu.sync_copy(sc0.at[pl.ds(0, rem)],
                                acc.at[pl.ds(base + q_full * _C, rem)])
            plsc.subcore_barrier()

            def blk(bi, _):
                b0 = bi * _BLK
                pltpu.sync_copy(src_hbm.at[p, cid, sid, pl.ds(b0, _BLK)], src_v)
                pltpu.sync_copy(dst_hbm.at[cid, sid, pl.ds(b0, _BLK)], dst_v)
                pltpu.sync_copy(w_hbm.at[cid, sid, pl.ds(b0, _BLK)], w_v)

                issue_gather(0, 0)
                issue_gather(1, 1)
                for par in (0, 1):
                    wait_gather(par)
                    scale(par, par)
                    issue_scatter(par, par)
                    issue_gather(par + 2, par)

                def pair(kk, _):
                    for par in (0, 1):
                        i = kk * 2 + par
                        wait_gather(par)
                        wait_scatter(par)
                        scale(i, par)
                        issue_scatter(i, par)
                        issue_gather(i + 2, par)
                    return 0
                lax.fori_loop(1, _BLK // 2 - 1, pair, 0)

                for par in (0, 1):
                    i = _BLK - 2 + par
                    wait_gather(par)
                    wait_scatter(par)
                    scale(i, par)
                    issue_scatter(i, par)
                wait_scatter(0)
                wait_scatter(1)
                return 0
            lax.fori_loop(0, _NB, blk, 0)
            plsc.subcore_barrier()

            @pl.when(sid < _WTILES)
            def _():
                pltpu.sync_copy(acc.at[pl.ds(base, _RPS)],
                                out_hbm.at[p, cid, pl.ds(base, _RPS)])
            plsc.subcore_barrier()

    return agg


_agg_16 = _make_agg(16, 1)
_agg_32 = _make_agg(32, 1)
_agg_64_1 = _make_agg(64, 1)
_agg_64_2 = _make_agg(64, 2)


def _stage0_body(degp_ref, x_ref, w1_ref, dinv_ref, h1_ref, hs1_ref):
    deg = degp_ref[:, 0:1] + 1.0
    dinv = lax.rsqrt(jnp.maximum(deg, 1e-12))
    dinv_ref[...] = dinv
    h1 = jnp.dot(x_ref[...], w1_ref[...], preferred_element_type=jnp.float32)
    h1_ref[...] = h1
    hs1_ref[...] = h1 * dinv


def _stage0(degp, x, w1):
    n, fo = x.shape[0], w1.shape[1]
    return pl.pallas_call(
        _stage0_body,
        out_shape=(
            jax.ShapeDtypeStruct((n, 1), jnp.float32),
            jax.ShapeDtypeStruct((n, fo), jnp.float32),
            jax.ShapeDtypeStruct((n, fo), jnp.float32),
        ),
    )(degp, x, w1)


def _combine(agg, h, dinv, b, g, be):
    out = dinv * agg + (dinv * dinv) * h + b
    m = jnp.mean(out, axis=0, keepdims=True)
    v = jnp.mean((out - m) ** 2, axis=0, keepdims=True)
    return jax.nn.relu((out - m) * lax.rsqrt(v + 1e-5) * g + be)


def _mid_body(agg_ref, h_ref, dinv_ref, b_ref, g_ref, be_ref, wn_ref,
              hn_ref, hsn_ref):
    dinv = dinv_ref[...]
    y = _combine(agg_ref[...], h_ref[...], dinv,
                 b_ref[...], g_ref[...], be_ref[...])
    hn = jnp.dot(y, wn_ref[...], preferred_element_type=jnp.float32)
    hn_ref[...] = hn
    hsn_ref[...] = hn * dinv


def _mid(agg, h, dinv, b, g, be, wn):
    n, fo = h.shape[0], wn.shape[1]
    return pl.pallas_call(
        _mid_body,
        out_shape=(
            jax.ShapeDtypeStruct((n, fo), jnp.float32),
            jax.ShapeDtypeStruct((n, fo), jnp.float32),
        ),
    )(agg, h, dinv, b, g, be, wn)


def _final_body(agg_ref, h_ref, dinv_ref, b_ref, g_ref, be_ref, batch_ref,
                wp1_ref, bp1_ref, wp2_ref, bp2_ref,
                pool_ref, z_ref, x3_ref):
    x3 = _combine(agg_ref[...], h_ref[...], dinv_ref[...],
                  b_ref[...], g_ref[...], be_ref[...])
    x3_ref[...] = x3
    gid = lax.broadcasted_iota(jnp.int32, (x3.shape[0], _G), 1)
    onehot = (batch_ref[...] == gid).astype(jnp.float32)
    s = lax.dot_general(onehot, x3, (((0,), (0,)), ((), ())),
                        preferred_element_type=jnp.float32)
    c = jnp.sum(onehot, axis=0)
    pool = s / jnp.maximum(c, 1.0)[:, None]
    pool_ref[...] = pool
    z1 = jax.nn.relu(
        jnp.dot(pool, wp1_ref[...], preferred_element_type=jnp.float32)
        + bp1_ref[...])
    z_ref[...] = (jnp.dot(z1, wp2_ref[...], preferred_element_type=jnp.float32)
                  + bp2_ref[...])


def _final(agg, h, dinv, b, g, be, batch2, wp1, bp1, wp2, bp2):
    n, f = h.shape
    return pl.pallas_call(
        _final_body,
        out_shape=(
            jax.ShapeDtypeStruct((_G, f), jnp.float32),
            jax.ShapeDtypeStruct((_G, f), jnp.float32),
            jax.ShapeDtypeStruct((n, f), jnp.float32),
        ),
    )(agg, h, dinv, b, g, be, batch2, wp1, bp1, wp2, bp2)


def _r1(a):
    return a.reshape(1, -1)


def kernel(x_t, edge_index_t, edge_attr_t, batch_t,
           x_f, edge_index_f, edge_attr_f, batch_f, params):
    p = params

    def eshape(a):
        return jnp.pad(a, (0, _EPAD - _E)).reshape(_NS, _PT, _C)

    src_all = jnp.stack([eshape(edge_index_t[0]),
                         eshape(edge_index_f[0]) + _N])
    dst_all = jnp.stack([eshape(edge_index_t[1]), eshape(edge_index_f[1])])
    w_all = jnp.stack([eshape(edge_attr_t), eshape(edge_attr_f)])
    src_1 = src_all[None]
    src_2 = jnp.stack([src_all, src_all + 2 * _N])

    ones_tab = jnp.ones((2 * _N, 16), jnp.float32)
    degp = _agg_16(ones_tab, src_1, dst_all, w_all)[0]

    dinv_t, h1_t, hs1_t = _stage0(degp[0], x_t, p['W1t'])
    dinv_f, h1_f, hs1_f = _stage0(degp[1], x_f, p['W1f'])

    stacked1 = jnp.concatenate([hs1_t, hs1_f])
    tab1 = stacked1.reshape(2 * _N, 2, 64).transpose(1, 0, 2).reshape(
        4 * _N, 64)
    agg1 = _agg_64_2(tab1, src_2, dst_all, w_all)
    agg1_t = jnp.concatenate([agg1[0, 0], agg1[1, 0]], axis=1)
    agg1_f = jnp.concatenate([agg1[0, 1], agg1[1, 1]], axis=1)

    h2_t, hs2_t = _mid(agg1_t, h1_t, dinv_t, _r1(p['b1t']),
                       _r1(p['g_bn1t']), _r1(p['be_bn1t']), p['W2t'])
    h2_f, hs2_f = _mid(agg1_f, h1_f, dinv_f, _r1(p['b1f']),
                       _r1(p['g_bn1f']), _r1(p['be_bn1f']), p['W2f'])

    tab2 = jnp.concatenate([hs2_t, hs2_f])
    agg2 = _agg_64_1(tab2, src_1, dst_all, w_all)[0]
    h3_t, hs3_t = _mid(agg2[0], h2_t, dinv_t, _r1(p['b2t']),
                       _r1(p['g_bn2t']), _r1(p['be_bn2t']), p['W3t'])
    h3_f, hs3_f = _mid(agg2[1], h2_f, dinv_f, _r1(p['b2f']),
                       _r1(p['g_bn2f']), _r1(p['be_bn2f']), p['W3f'])

    tab3 = jnp.concatenate([hs3_t, hs3_f])
    agg3 = _agg_32(tab3, src_1, dst_all, w_all)[0]

    h_time, z_time, xt = _final(
        agg3[0], h3_t, dinv_t, _r1(p['b3t']), _r1(p['g_bn3t']),
        _r1(p['be_bn3t']), batch_t.reshape(_N, 1),
        p['Wp1t'], _r1(p['bp1t']), p['Wp2t'], _r1(p['bp2t']))
    h_freq, z_freq, xf = _final(
        agg3[1], h3_f, dinv_f, _r1(p['b3f']), _r1(p['g_bn3f']),
        _r1(p['be_bn3f']), batch_f.reshape(_N, 1),
        p['Wp1f'], _r1(p['bp1f']), p['Wp2f'], _r1(p['bp2f']))

    return (h_time, z_time, h_freq, z_freq, xt, xf)


# speedup vs baseline: 1.6980x; 1.2190x over previous
import functools

import jax
import jax.numpy as jnp
from jax import lax
from jax.experimental import pallas as pl
from jax.experimental.pallas import tpu as pltpu
from jax.experimental.pallas import tpu_sc as plsc

_N = 10000
_E = 640000
_G = 64
_C = 80
_NC = 2
_NS = 16
_PT = 500
_EPAD = _NS * _PT * _C
_CHUNKS = _EPAD // _C
_BLK = 50
_NB = _PT // _BLK
_WTILES = 10
_RPS = _N // _WTILES

_mesh = plsc.VectorSubcoreMesh(
    core_axis_name="c", subcore_axis_name="s",
    num_cores=_NC, num_subcores=_NS)


def _make_agg(F, NPASS):
    @functools.partial(
        pl.kernel,
        out_type=jax.ShapeDtypeStruct((NPASS, _NC, _N, F), jnp.float32),
        mesh=_mesh,
        scratch_types=[
            pltpu.VMEM((_BLK, _C), jnp.int32),
            pltpu.VMEM((_BLK, _C), jnp.int32),
            pltpu.VMEM((_BLK, _C), jnp.float32),
            pltpu.VMEM((_C, F), jnp.float32),
            pltpu.VMEM((_C, F), jnp.float32),
            pltpu.VMEM((_C, F), jnp.float32),
            pltpu.VMEM((_C, F), jnp.float32),
            pltpu.VMEM_SHARED((_N, F), jnp.float32),
            pltpu.SemaphoreType.DMA,
            pltpu.SemaphoreType.DMA,
            pltpu.SemaphoreType.DMA,
            pltpu.SemaphoreType.DMA,
        ],
        compiler_params=pltpu.CompilerParams(use_tc_tiling_on_sc=False),
    )
    def agg(tab_hbm, src_hbm, dst_hbm, w_hbm, out_hbm,
            src_v, dst_v, w_v, g0, g1, sc0, sc1, acc,
            gs0, gs1, ss0, ss1):
        cid = lax.axis_index("c")
        sid = lax.axis_index("s")
        base = sid * _RPS
        q_full = _RPS // _C
        rem = _RPS - q_full * _C
        GB, SB = (g0, g1), (sc0, sc1)
        GS, SS = (gs0, gs1), (ss0, ss1)

        def issue_gather(i, par):
            pltpu.async_copy(tab_hbm.at[src_v.at[i]], GB[par], GS[par])

        def wait_gather(par):
            pltpu.make_async_copy(tab_hbm.at[src_v.at[0]],
                                  GB[par], GS[par]).wait()

        def issue_scatter(i, par):
            pltpu.async_copy(SB[par], acc.at[dst_v.at[i]], SS[par], add=True)

        def wait_scatter(par):
            pltpu.make_async_copy(SB[par], acc.at[pl.ds(0, _C)],
                                  SS[par]).wait()

        def scale(i, par):
            gbuf, sbuf = GB[par], SB[par]

            def rowblk(rb, _):
                r0 = rb * 16
                wvec = w_v[i, pl.ds(r0, 16)]
                for t in range(16):
                    wv = wvec[t]
                    for k in range(F // 16):
                        sl = pl.ds(k * 16, 16)
                        sbuf[r0 + t, sl] = gbuf[r0 + t, sl] * wv
                return 0
            lax.fori_loop(0, _C // 16, rowblk, 0)

        for p in range(NPASS):
            def zr(r, _):
                for k in range(F // 16):
                    sc0[r, pl.ds(k * 16, 16)] = jnp.zeros((16,), jnp.float32)
                return 0
            lax.fori_loop(0, _C, zr, 0)

            @pl.when(sid < _WTILES)
            def _():
                for q in range(q_full):
                    pltpu.sync_copy(sc0, acc.at[pl.ds(base + q * _C, _C)])
                pltpu.sync_copy(sc0.at[pl.ds(0, rem)],
                                acc.at[pl.ds(base + q_full * _C, rem)])
            plsc.subcore_barrier()

            def blk(bi, _):
                b0 = bi * _BLK
                pltpu.sync_copy(src_hbm.at[p, cid, sid, pl.ds(b0, _BLK)], src_v)
                pltpu.sync_copy(dst_hbm.at[cid, sid, pl.ds(b0, _BLK)], dst_v)
                pltpu.sync_copy(w_hbm.at[cid, sid, pl.ds(b0, _BLK)], w_v)

                issue_gather(0, 0)
                issue_gather(1, 1)
                for par in (0, 1):
                    wait_gather(par)
                    scale(par, par)
                    issue_scatter(par, par)
                    issue_gather(par + 2, par)

                def pair(kk, _):
                    for par in (0, 1):
                        i = kk * 2 + par
                        wait_gather(par)
                        wait_scatter(par)
                        scale(i, par)
                        issue_scatter(i, par)
                        issue_gather(i + 2, par)
                    return 0
                lax.fori_loop(1, _BLK // 2 - 1, pair, 0)

                for par in (0, 1):
                    i = _BLK - 2 + par
                    wait_gather(par)
                    wait_scatter(par)
                    scale(i, par)
                    issue_scatter(i, par)
                wait_scatter(0)
                wait_scatter(1)
                return 0
            lax.fori_loop(0, _NB, blk, 0)
            plsc.subcore_barrier()

            @pl.when(sid < _WTILES)
            def _():
                pltpu.sync_copy(acc.at[pl.ds(base, _RPS)],
                                out_hbm.at[p, cid, pl.ds(base, _RPS)])
            plsc.subcore_barrier()

    return agg


_agg_16 = _make_agg(16, 1)
_agg_32 = _make_agg(32, 1)
_agg_64_1 = _make_agg(64, 1)
_agg_64_2 = _make_agg(64, 2)


def _stage0_body(degp_ref, xt_ref, xf_ref, w1t_ref, w1f_ref,
                 dinv_ref, tab1_ref):
    for b, (x_ref, w_ref) in enumerate(((xt_ref, w1t_ref),
                                        (xf_ref, w1f_ref))):
        deg = degp_ref[b, :, 0:1] + 1.0
        dinv = lax.rsqrt(jnp.maximum(deg, 1e-12))
        dinv_ref[b] = dinv
        h1 = jnp.dot(x_ref[...], w_ref[...],
                     preferred_element_type=jnp.float32)
        hs = h1 * dinv
        tab1_ref[0, pl.ds(b * _N, _N)] = hs[:, 0:64]
        tab1_ref[1, pl.ds(b * _N, _N)] = hs[:, 64:128]


def _stage0(degp, x_t, x_f, w1t, w1f):
    return pl.pallas_call(
        _stage0_body,
        compiler_params=pltpu.CompilerParams(
            vmem_limit_bytes=100 * 1024 * 1024),
        out_shape=(
            jax.ShapeDtypeStruct((2, _N, 1), jnp.float32),
            jax.ShapeDtypeStruct((2, 2 * _N, 64), jnp.float32),
        ),
    )(degp, x_t, x_f, w1t, w1f)


def _bn_relu(out, g, be):
    m = jnp.mean(out, axis=0, keepdims=True)
    v = jnp.mean((out - m) ** 2, axis=0, keepdims=True)
    return jax.nn.relu((out - m) * lax.rsqrt(v + 1e-5) * g + be)


def _mid_split_body(agg_ref, tab_ref, dinv_ref, pb_ref, pg_ref, pbe_ref,
                    w_ref, tabn_ref):
    o = dinv_ref[0] * (agg_ref[0, 0] + tab_ref[0]) + pb_ref[0, 0]
    y = _bn_relu(o, pg_ref[0, 0], pbe_ref[0, 0])
    part = jnp.dot(y, w_ref[0], preferred_element_type=jnp.float32)
    p2 = pl.program_id(1)

    @pl.when(p2 == 0)
    def _():
        tabn_ref[...] = part

    @pl.when(p2 == 1)
    def _():
        tabn_ref[...] = (tabn_ref[...] + part) * dinv_ref[0]


def _mid_body(agg_ref, tab_ref, dinv_ref, pb_ref, pg_ref, pbe_ref,
              w_ref, tabn_ref):
    dinv = dinv_ref[0]
    o = dinv * (agg_ref[0] + tab_ref[...]) + pb_ref[0]
    y = _bn_relu(o, pg_ref[0], pbe_ref[0])
    hn = jnp.dot(y, w_ref[0], preferred_element_type=jnp.float32)
    tabn_ref[...] = hn * dinv


def _mid(split, agg, tab, dinv, pb, pg, pbe, w2):
    fi, fo = w2.shape[1], w2.shape[2]
    if split:
        return pl.pallas_call(
            _mid_split_body,
            grid=(2, 2),
            in_specs=[
                pl.BlockSpec((1, 1, _N, 64), lambda b, q: (q, b, 0, 0)),
                pl.BlockSpec((1, _N, 64), lambda b, q: (q, b, 0)),
                pl.BlockSpec((1, _N, 1), lambda b, q: (b, 0, 0)),
                pl.BlockSpec((1, 1, 1, 64), lambda b, q: (b, q, 0, 0)),
                pl.BlockSpec((1, 1, 1, 64), lambda b, q: (b, q, 0, 0)),
                pl.BlockSpec((1, 1, 1, 64), lambda b, q: (b, q, 0, 0)),
                pl.BlockSpec((1, 64, fo), lambda b, q: (b, q, 0)),
            ],
            out_specs=pl.BlockSpec((_N, fo), lambda b, q: (b, 0)),
            compiler_params=pltpu.CompilerParams(
                vmem_limit_bytes=100 * 1024 * 1024),
            out_shape=jax.ShapeDtypeStruct((2 * _N, fo), jnp.float32),
        )(agg, tab, dinv, pb, pg, pbe, w2)
    return pl.pallas_call(
        _mid_body,
        grid=(2,),
        in_specs=[
            pl.BlockSpec((1, _N, fi), lambda b: (b, 0, 0)),
            pl.BlockSpec((_N, fi), lambda b: (b, 0)),
            pl.BlockSpec((1, _N, 1), lambda b: (b, 0, 0)),
            pl.BlockSpec((1, 1, fi), lambda b: (b, 0, 0)),
            pl.BlockSpec((1, 1, fi), lambda b: (b, 0, 0)),
            pl.BlockSpec((1, 1, fi), lambda b: (b, 0, 0)),
            pl.BlockSpec((1, fi, fo), lambda b: (b, 0, 0)),
        ],
        out_specs=pl.BlockSpec((_N, fo), lambda b: (b, 0)),
        compiler_params=pltpu.CompilerParams(
            vmem_limit_bytes=100 * 1024 * 1024),
        out_shape=jax.ShapeDtypeStruct((2 * _N, fo), jnp.float32),
    )(agg, tab, dinv, pb, pg, pbe, w2)


def _final_body(agg_ref, tab_ref, dinv_ref, pb_ref, pg_ref, pbe_ref,
                batch_ref, wp1_ref, bp1_ref, wp2_ref, bp2_ref,
                pool_ref, z_ref, x3_ref):
    o = (dinv_ref[0] * (agg_ref[0] + tab_ref[...]) + pb_ref[0])
    x3 = _bn_relu(o, pg_ref[0], pbe_ref[0])
    x3_ref[0] = x3
    gid = lax.broadcasted_iota(jnp.int32, (_N, _G), 1)
    onehot = (batch_ref[0] == gid).astype(jnp.float32)
    s = lax.dot_general(onehot, x3, (((0,), (0,)), ((), ())),
                        preferred_element_type=jnp.float32)
    c = jnp.sum(onehot, axis=0)
    pool = s / jnp.maximum(c, 1.0)[:, None]
    pool_ref[0] = pool
    z1 = jax.nn.relu(
        jnp.dot(pool, wp1_ref[0], preferred_element_type=jnp.float32)
        + bp1_ref[0])
    z_ref[0] = (jnp.dot(z1, wp2_ref[0],
                        preferred_element_type=jnp.float32)
                + bp2_ref[0])


def _final(agg, tab, dinv, pb, pg, pbe, batch2, wp1, bp1, wp2, bp2):
    pool, z, x3 = pl.pallas_call(
        _final_body,
        grid=(2,),
        in_specs=[
            pl.BlockSpec((1, _N, 32), lambda b: (b, 0, 0)),
            pl.BlockSpec((_N, 32), lambda b: (b, 0)),
            pl.BlockSpec((1, _N, 1), lambda b: (b, 0, 0)),
            pl.BlockSpec((1, 1, 32), lambda b: (b, 0, 0)),
            pl.BlockSpec((1, 1, 32), lambda b: (b, 0, 0)),
            pl.BlockSpec((1, 1, 32), lambda b: (b, 0, 0)),
            pl.BlockSpec((1, _N, 1), lambda b: (b, 0, 0)),
            pl.BlockSpec((1, 32, 64), lambda b: (b, 0, 0)),
            pl.BlockSpec((1, 1, 64), lambda b: (b, 0, 0)),
            pl.BlockSpec((1, 64, 32), lambda b: (b, 0, 0)),
            pl.BlockSpec((1, 1, 32), lambda b: (b, 0, 0)),
        ],
        out_specs=(
            pl.BlockSpec((1, _G, 32), lambda b: (b, 0, 0)),
            pl.BlockSpec((1, _G, 32), lambda b: (b, 0, 0)),
            pl.BlockSpec((1, _N, 32), lambda b: (b, 0, 0)),
        ),
        compiler_params=pltpu.CompilerParams(
            vmem_limit_bytes=100 * 1024 * 1024),
        out_shape=(
            jax.ShapeDtypeStruct((2, _G, 32), jnp.float32),
            jax.ShapeDtypeStruct((2, _G, 32), jnp.float32),
            jax.ShapeDtypeStruct((2, _N, 32), jnp.float32),
        ),
    )(agg, tab, dinv, pb, pg, pbe, batch2, wp1, bp1, wp2, bp2)
    return pool[0], z[0], pool[1], z[1], x3[0], x3[1]


def _pstack(pt, pf):
    return jnp.stack([pt.reshape(1, -1), pf.reshape(1, -1)])


def _pstack4(pt, pf):
    return jnp.stack([pt.reshape(2, 1, 64), pf.reshape(2, 1, 64)])


def kernel(x_t, edge_index_t, edge_attr_t, batch_t,
           x_f, edge_index_f, edge_attr_f, batch_f, params):
    p = params

    def eshape(a):
        return jnp.pad(a, (0, _EPAD - _E)).reshape(_NS, _PT, _C)

    src_all = jnp.stack([eshape(edge_index_t[0]),
                         eshape(edge_index_f[0]) + _N])
    dst_all = jnp.stack([eshape(edge_index_t[1]), eshape(edge_index_f[1])])
    w_all = jnp.stack([eshape(edge_attr_t), eshape(edge_attr_f)])
    src_1 = src_all[None]
    src_2 = jnp.stack([src_all, src_all + 2 * _N])

    ones_tab = jnp.ones((2 * _N, 16), jnp.float32)
    degp = _agg_16(ones_tab, src_1, dst_all, w_all)[0]

    dinv, tab1 = _stage0(degp, x_t, x_f, p['W1t'], p['W1f'])

    agg1 = _agg_64_2(tab1.reshape(4 * _N, 64), src_2, dst_all, w_all)
    tab2 = _mid(True, agg1, tab1, dinv, _pstack4(p['b1t'], p['b1f']),
                _pstack4(p['g_bn1t'], p['g_bn1f']),
                _pstack4(p['be_bn1t'], p['be_bn1f']),
                jnp.stack([p['W2t'], p['W2f']]))

    agg2 = _agg_64_1(tab2, src_1, dst_all, w_all)[0]
    tab3 = _mid(False, agg2, tab2, dinv, _pstack(p['b2t'], p['b2f']),
                _pstack(p['g_bn2t'], p['g_bn2f']),
                _pstack(p['be_bn2t'], p['be_bn2f']),
                jnp.stack([p['W3t'], p['W3f']]))

    agg3 = _agg_32(tab3, src_1, dst_all, w_all)[0]
    batch2 = jnp.stack([batch_t.reshape(_N, 1), batch_f.reshape(_N, 1)])
    h_time, z_time, h_freq, z_freq, xt, xf = _final(
        agg3, tab3, dinv, _pstack(p['b3t'], p['b3f']),
        _pstack(p['g_bn3t'], p['g_bn3f']), _pstack(p['be_bn3t'], p['be_bn3f']),
        batch2, jnp.stack([p['Wp1t'], p['Wp1f']]),
        _pstack(p['bp1t'], p['bp1f']), jnp.stack([p['Wp2t'], p['Wp2f']]),
        _pstack(p['bp2t'], p['bp2f']))

    return (h_time, z_time, h_freq, z_freq, xt, xf)


# speedup vs baseline: 1.9234x; 1.1327x over previous
import functools

import jax
import jax.numpy as jnp
from jax import lax
from jax.experimental import pallas as pl
from jax.experimental.pallas import tpu as pltpu
from jax.experimental.pallas import tpu_sc as plsc

_N = 10000
_E = 640000
_G = 64
_C = 80
_NC = 2
_NS = 16
_PT = 500
_EPAD = _NS * _PT * _C
_CHUNKS = _EPAD // _C
_BLK = 50
_NB = _PT // _BLK
_WTILES = 10
_RPS = _N // _WTILES

_mesh = plsc.VectorSubcoreMesh(
    core_axis_name="c", subcore_axis_name="s",
    num_cores=_NC, num_subcores=_NS)


def _make_agg(F, NPASS):
    @functools.partial(
        pl.kernel,
        out_type=jax.ShapeDtypeStruct((NPASS, _NC, _N, F), jnp.float32),
        mesh=_mesh,
        scratch_types=[
            pltpu.VMEM((_BLK, _C), jnp.int32),
            pltpu.VMEM((_BLK, _C), jnp.int32),
            pltpu.VMEM((_BLK, _C), jnp.float32),
            pltpu.VMEM((_C, F), jnp.float32),
            pltpu.VMEM((_C, F), jnp.float32),
            pltpu.VMEM((_C, F), jnp.float32),
            pltpu.VMEM((_C, F), jnp.float32),
            pltpu.VMEM_SHARED((_N, F), jnp.float32),
            pltpu.SemaphoreType.DMA,
            pltpu.SemaphoreType.DMA,
            pltpu.SemaphoreType.DMA,
            pltpu.SemaphoreType.DMA,
        ],
        compiler_params=pltpu.CompilerParams(use_tc_tiling_on_sc=False),
    )
    def agg(tab_hbm, src_hbm, dst_hbm, w_hbm, out_hbm,
            src_v, dst_v, w_v, g0, g1, sc0, sc1, acc,
            gs0, gs1, ss0, ss1):
        cid = lax.axis_index("c")
        sid = lax.axis_index("s")
        base = sid * _RPS
        q_full = _RPS // _C
        rem = _RPS - q_full * _C
        GB, SB = (g0, g1), (sc0, sc1)
        GS, SS = (gs0, gs1), (ss0, ss1)

        def issue_gather(i, par):
            pltpu.async_copy(tab_hbm.at[src_v.at[i]], GB[par], GS[par])

        def wait_gather(par):
            pltpu.make_async_copy(tab_hbm.at[src_v.at[0]],
                                  GB[par], GS[par]).wait()

        def issue_scatter(i, par):
            pltpu.async_copy(SB[par], acc.at[dst_v.at[i]], SS[par], add=True)

        def wait_scatter(par):
            pltpu.make_async_copy(SB[par], acc.at[pl.ds(0, _C)],
                                  SS[par]).wait()

        def scale(i, par):
            gbuf, sbuf = GB[par], SB[par]

            def rowblk(rb, _):
                r0 = rb * 16
                wvec = w_v[i, pl.ds(r0, 16)]
                for t in range(16):
                    wv = wvec[t]
                    for k in range(F // 16):
                        sl = pl.ds(k * 16, 16)
                        sbuf[r0 + t, sl] = gbuf[r0 + t, sl] * wv
                return 0
            lax.fori_loop(0, _C // 16, rowblk, 0)

        for p in range(NPASS):
            def zr(r, _):
                for k in range(F // 16):
                    sc0[r, pl.ds(k * 16, 16)] = jnp.zeros((16,), jnp.float32)
                return 0
            lax.fori_loop(0, _C, zr, 0)

            @pl.when(sid < _WTILES)
            def _():
                for q in range(q_full):
                    pltpu.sync_copy(sc0, acc.at[pl.ds(base + q * _C, _C)])
                pltpu.sync_copy(sc0.at[pl.ds(0, rem)],
                                acc.at[pl.ds(base + q_full * _C, rem)])
            plsc.subcore_barrier()

            def blk(bi, _):
                b0 = bi * _BLK
                pltpu.sync_copy(src_hbm.at[p, cid, sid, pl.ds(b0, _BLK)], src_v)
                pltpu.sync_copy(dst_hbm.at[cid, sid, pl.ds(b0, _BLK)], dst_v)
                pltpu.sync_copy(w_hbm.at[cid, sid, pl.ds(b0, _BLK)], w_v)

                issue_gather(0, 0)
                issue_gather(1, 1)
                for par in (0, 1):
                    wait_gather(par)
                    scale(par, par)
                    issue_scatter(par, par)
                    issue_gather(par + 2, par)

                def pair(kk, _):
                    for par in (0, 1):
                        i = kk * 2 + par
                        wait_gather(par)
                        wait_scatter(par)
                        scale(i, par)
                        issue_scatter(i, par)
                        issue_gather(i + 2, par)
                    return 0
                lax.fori_loop(1, _BLK // 2 - 1, pair, 0)

                for par in (0, 1):
                    i = _BLK - 2 + par
                    wait_gather(par)
                    wait_scatter(par)
                    scale(i, par)
                    issue_scatter(i, par)
                wait_scatter(0)
                wait_scatter(1)
                return 0
            lax.fori_loop(0, _NB, blk, 0)
            plsc.subcore_barrier()

            @pl.when(sid < _WTILES)
            def _():
                pltpu.sync_copy(acc.at[pl.ds(base, _RPS)],
                                out_hbm.at[p, cid, pl.ds(base, _RPS)])
            plsc.subcore_barrier()

    return agg


@functools.partial(
    pl.kernel,
    out_type=jax.ShapeDtypeStruct((_NC, _NS, _N), jnp.float32),
    mesh=_mesh,
    scratch_types=[
        pltpu.VMEM((_BLK, _C), jnp.int32),
        pltpu.VMEM((_BLK, _C), jnp.float32),
        pltpu.VMEM((_N,), jnp.float32),
    ],
    compiler_params=pltpu.CompilerParams(use_tc_tiling_on_sc=False,
                                         needs_layout_passes=False),
)
def _deg(dst_hbm, w_hbm, out_hbm, dst_v, w_v, deg_v):
    cid = lax.axis_index("c")
    sid = lax.axis_index("s")

    def zr(r, _):
        deg_v[pl.ds(r * 16, 16)] = jnp.zeros((16,), jnp.float32)
        return 0
    lax.fori_loop(0, _N // 16, zr, 0)

    def blk(bi, _):
        b0 = bi * _BLK
        pltpu.sync_copy(dst_hbm.at[cid, sid, pl.ds(b0, _BLK)], dst_v)
        pltpu.sync_copy(w_hbm.at[cid, sid, pl.ds(b0, _BLK)], w_v)

        def chunk(i, _):
            for v in range(_C // 16):
                sl = pl.ds(v * 16, 16)
                plsc.addupdate_scatter(deg_v, [dst_v[i, sl]], w_v[i, sl])
            return 0
        lax.fori_loop(0, _BLK, chunk, 0)
        return 0
    lax.fori_loop(0, _NB, blk, 0)
    pltpu.sync_copy(deg_v, out_hbm.at[cid, sid])


_agg_32 = _make_agg(32, 1)
_agg_64_1 = _make_agg(64, 1)
_agg_64_2 = _make_agg(64, 2)


def _stage0_body(degp_ref, xt_ref, xf_ref, w1t_ref, w1f_ref,
                 dinv_ref, tab1_ref):
    for b, (x_ref, w_ref) in enumerate(((xt_ref, w1t_ref),
                                        (xf_ref, w1f_ref))):
        deg = jnp.sum(degp_ref[b], axis=0).reshape(_N, 1) + 1.0
        dinv = lax.rsqrt(jnp.maximum(deg, 1e-12))
        dinv_ref[b] = dinv
        h1 = jnp.dot(x_ref[...], w_ref[...],
                     preferred_element_type=jnp.float32)
        hs = h1 * dinv
        tab1_ref[0, pl.ds(b * _N, _N)] = hs[:, 0:64]
        tab1_ref[1, pl.ds(b * _N, _N)] = hs[:, 64:128]


def _stage0(degp, x_t, x_f, w1t, w1f):
    return pl.pallas_call(
        _stage0_body,
        compiler_params=pltpu.CompilerParams(
            vmem_limit_bytes=100 * 1024 * 1024),
        out_shape=(
            jax.ShapeDtypeStruct((2, _N, 1), jnp.float32),
            jax.ShapeDtypeStruct((2, 2 * _N, 64), jnp.float32),
        ),
    )(degp, x_t, x_f, w1t, w1f)


def _bn_relu(out, g, be):
    m = jnp.mean(out, axis=0, keepdims=True)
    v = jnp.mean((out - m) ** 2, axis=0, keepdims=True)
    return jax.nn.relu((out - m) * lax.rsqrt(v + 1e-5) * g + be)


def _mid_split_body(agg_ref, tab_ref, dinv_ref, pb_ref, pg_ref, pbe_ref,
                    w_ref, tabn_ref):
    o = dinv_ref[0] * (agg_ref[0, 0] + tab_ref[0]) + pb_ref[0, 0]
    y = _bn_relu(o, pg_ref[0, 0], pbe_ref[0, 0])
    part = jnp.dot(y, w_ref[0], preferred_element_type=jnp.float32)
    p2 = pl.program_id(1)

    @pl.when(p2 == 0)
    def _():
        tabn_ref[...] = part

    @pl.when(p2 == 1)
    def _():
        tabn_ref[...] = (tabn_ref[...] + part) * dinv_ref[0]


def _mid_body(agg_ref, tab_ref, dinv_ref, pb_ref, pg_ref, pbe_ref,
              w_ref, tabn_ref):
    dinv = dinv_ref[0]
    o = dinv * (agg_ref[0] + tab_ref[...]) + pb_ref[0]
    y = _bn_relu(o, pg_ref[0], pbe_ref[0])
    hn = jnp.dot(y, w_ref[0], preferred_element_type=jnp.float32)
    tabn_ref[...] = hn * dinv


def _mid(split, agg, tab, dinv, pb, pg, pbe, w2):
    fi, fo = w2.shape[1], w2.shape[2]
    if split:
        return pl.pallas_call(
            _mid_split_body,
            grid=(2, 2),
            in_specs=[
                pl.BlockSpec((1, 1, _N, 64), lambda b, q: (q, b, 0, 0)),
                pl.BlockSpec((1, _N, 64), lambda b, q: (q, b, 0)),
                pl.BlockSpec((1, _N, 1), lambda b, q: (b, 0, 0)),
                pl.BlockSpec((1, 1, 1, 64), lambda b, q: (b, q, 0, 0)),
                pl.BlockSpec((1, 1, 1, 64), lambda b, q: (b, q, 0, 0)),
                pl.BlockSpec((1, 1, 1, 64), lambda b, q: (b, q, 0, 0)),
                pl.BlockSpec((1, 64, fo), lambda b, q: (b, q, 0)),
            ],
            out_specs=pl.BlockSpec((_N, fo), lambda b, q: (b, 0)),
            compiler_params=pltpu.CompilerParams(
                vmem_limit_bytes=100 * 1024 * 1024),
            out_shape=jax.ShapeDtypeStruct((2 * _N, fo), jnp.float32),
        )(agg, tab, dinv, pb, pg, pbe, w2)
    return pl.pallas_call(
        _mid_body,
        grid=(2,),
        in_specs=[
            pl.BlockSpec((1, _N, fi), lambda b: (b, 0, 0)),
            pl.BlockSpec((_N, fi), lambda b: (b, 0)),
            pl.BlockSpec((1, _N, 1), lambda b: (b, 0, 0)),
            pl.BlockSpec((1, 1, fi), lambda b: (b, 0, 0)),
            pl.BlockSpec((1, 1, fi), lambda b: (b, 0, 0)),
            pl.BlockSpec((1, 1, fi), lambda b: (b, 0, 0)),
            pl.BlockSpec((1, fi, fo), lambda b: (b, 0, 0)),
        ],
        out_specs=pl.BlockSpec((_N, fo), lambda b: (b, 0)),
        compiler_params=pltpu.CompilerParams(
            vmem_limit_bytes=100 * 1024 * 1024),
        out_shape=jax.ShapeDtypeStruct((2 * _N, fo), jnp.float32),
    )(agg, tab, dinv, pb, pg, pbe, w2)


def _final_body(agg_ref, tab_ref, dinv_ref, pb_ref, pg_ref, pbe_ref,
                batch_ref, wp1_ref, bp1_ref, wp2_ref, bp2_ref,
                pool_ref, z_ref, x3_ref):
    o = (dinv_ref[0] * (agg_ref[0] + tab_ref[...]) + pb_ref[0])
    x3 = _bn_relu(o, pg_ref[0], pbe_ref[0])
    x3_ref[0] = x3
    gid = lax.broadcasted_iota(jnp.int32, (_N, _G), 1)
    onehot = (batch_ref[0] == gid).astype(jnp.float32)
    s = lax.dot_general(onehot, x3, (((0,), (0,)), ((), ())),
                        preferred_element_type=jnp.float32)
    c = jnp.sum(onehot, axis=0)
    pool = s / jnp.maximum(c, 1.0)[:, None]
    pool_ref[0] = pool
    z1 = jax.nn.relu(
        jnp.dot(pool, wp1_ref[0], preferred_element_type=jnp.float32)
        + bp1_ref[0])
    z_ref[0] = (jnp.dot(z1, wp2_ref[0],
                        preferred_element_type=jnp.float32)
                + bp2_ref[0])


def _final(agg, tab, dinv, pb, pg, pbe, batch2, wp1, bp1, wp2, bp2):
    pool, z, x3 = pl.pallas_call(
        _final_body,
        grid=(2,),
        in_specs=[
            pl.BlockSpec((1, _N, 32), lambda b: (b, 0, 0)),
            pl.BlockSpec((_N, 32), lambda b: (b, 0)),
            pl.BlockSpec((1, _N, 1), lambda b: (b, 0, 0)),
            pl.BlockSpec((1, 1, 32), lambda b: (b, 0, 0)),
            pl.BlockSpec((1, 1, 32), lambda b: (b, 0, 0)),
            pl.BlockSpec((1, 1, 32), lambda b: (b, 0, 0)),
            pl.BlockSpec((1, _N, 1), lambda b: (b, 0, 0)),
            pl.BlockSpec((1, 32, 64), lambda b: (b, 0, 0)),
            pl.BlockSpec((1, 1, 64), lambda b: (b, 0, 0)),
            pl.BlockSpec((1, 64, 32), lambda b: (b, 0, 0)),
            pl.BlockSpec((1, 1, 32), lambda b: (b, 0, 0)),
        ],
        out_specs=(
            pl.BlockSpec((1, _G, 32), lambda b: (b, 0, 0)),
            pl.BlockSpec((1, _G, 32), lambda b: (b, 0, 0)),
            pl.BlockSpec((1, _N, 32), lambda b: (b, 0, 0)),
        ),
        compiler_params=pltpu.CompilerParams(
            vmem_limit_bytes=100 * 1024 * 1024),
        out_shape=(
            jax.ShapeDtypeStruct((2, _G, 32), jnp.float32),
            jax.ShapeDtypeStruct((2, _G, 32), jnp.float32),
            jax.ShapeDtypeStruct((2, _N, 32), jnp.float32),
        ),
    )(agg, tab, dinv, pb, pg, pbe, batch2, wp1, bp1, wp2, bp2)
    return pool[0], z[0], pool[1], z[1], x3[0], x3[1]


def _pstack(pt, pf):
    return jnp.stack([pt.reshape(1, -1), pf.reshape(1, -1)])


def _pstack4(pt, pf):
    return jnp.stack([pt.reshape(2, 1, 64), pf.reshape(2, 1, 64)])


def kernel(x_t, edge_index_t, edge_attr_t, batch_t,
           x_f, edge_index_f, edge_attr_f, batch_f, params):
    p = params

    def eshape(a):
        return jnp.pad(a, (0, _EPAD - _E)).reshape(_NS, _PT, _C)

    src_all = jnp.stack([eshape(edge_index_t[0]),
                         eshape(edge_index_f[0]) + _N])
    dst_all = jnp.stack([eshape(edge_index_t[1]), eshape(edge_index_f[1])])
    w_all = jnp.stack([eshape(edge_attr_t), eshape(edge_attr_f)])
    src_1 = src_all[None]
    src_2 = jnp.stack([src_all, src_all + 2 * _N])

    degp = _deg(dst_all, w_all)

    dinv, tab1 = _stage0(degp, x_t, x_f, p['W1t'], p['W1f'])

    agg1 = _agg_64_2(tab1.reshape(4 * _N, 64), src_2, dst_all, w_all)
    tab2 = _mid(True, agg1, tab1, dinv, _pstack4(p['b1t'], p['b1f']),
                _pstack4(p['g_bn1t'], p['g_bn1f']),
                _pstack4(p['be_bn1t'], p['be_bn1f']),
                jnp.stack([p['W2t'], p['W2f']]))

    agg2 = _agg_64_1(tab2, src_1, dst_all, w_all)[0]
    tab3 = _mid(False, agg2, tab2, dinv, _pstack(p['b2t'], p['b2f']),
                _pstack(p['g_bn2t'], p['g_bn2f']),
                _pstack(p['be_bn2t'], p['be_bn2f']),
                jnp.stack([p['W3t'], p['W3f']]))

    agg3 = _agg_32(tab3, src_1, dst_all, w_all)[0]
    batch2 = jnp.stack([batch_t.reshape(_N, 1), batch_f.reshape(_N, 1)])
    h_time, z_time, h_freq, z_freq, xt, xf = _final(
        agg3, tab3, dinv, _pstack(p['b3t'], p['b3f']),
        _pstack(p['g_bn3t'], p['g_bn3f']), _pstack(p['be_bn3t'], p['be_bn3f']),
        batch2, jnp.stack([p['Wp1t'], p['Wp1f']]),
        _pstack(p['bp1t'], p['bp1f']), jnp.stack([p['Wp2t'], p['Wp2f']]),
        _pstack(p['bp2t'], p['bp2f']))

    return (h_time, z_time, h_freq, z_freq, xt, xf)


# speedup vs baseline: 1.9950x; 1.0373x over previous
import functools

import jax
import jax.numpy as jnp
from jax import lax
from jax.experimental import pallas as pl
from jax.experimental.pallas import tpu as pltpu
from jax.experimental.pallas import tpu_sc as plsc

_N = 10000
_E = 640000
_G = 64
_C = 80
_NC = 2
_NS = 16
_PT = 500
_EPAD = _NS * _PT * _C
_CHUNKS = _EPAD // _C
_BLK = 50
_NB = _PT // _BLK
_WTILES = 10
_RPS = _N // _WTILES

_mesh = plsc.VectorSubcoreMesh(
    core_axis_name="c", subcore_axis_name="s",
    num_cores=_NC, num_subcores=_NS)


def _make_agg(F, NPASS, pipelined):
    NEB = 2 if pipelined else 1

    @functools.partial(
        pl.kernel,
        out_type=jax.ShapeDtypeStruct((NPASS, _NC, _N, F), jnp.float32),
        mesh=_mesh,
        scratch_types=[
            pltpu.VMEM((NEB, _BLK, _C), jnp.int32),
            pltpu.VMEM((NEB, _BLK, _C), jnp.int32),
            pltpu.VMEM((NEB, _BLK, _C), jnp.float32),
            pltpu.VMEM((_C, F), jnp.float32),
            pltpu.VMEM((_C, F), jnp.float32),
            pltpu.VMEM((_C, F), jnp.float32),
            pltpu.VMEM((_C, F), jnp.float32),
            pltpu.VMEM_SHARED((_N, F), jnp.float32),
            pltpu.SemaphoreType.DMA,
            pltpu.SemaphoreType.DMA,
            pltpu.SemaphoreType.DMA,
            pltpu.SemaphoreType.DMA,
            pltpu.SemaphoreType.DMA,
            pltpu.SemaphoreType.DMA,
        ],
        compiler_params=pltpu.CompilerParams(use_tc_tiling_on_sc=False),
    )
    def agg(tab_hbm, src_hbm, dst_hbm, w_hbm, out_hbm,
            src_v, dst_v, w_v, g0, g1, sc0, sc1, acc,
            gs0, gs1, ss0, ss1, es0, es1):
        cid = lax.axis_index("c")
        sid = lax.axis_index("s")
        base = sid * _RPS
        q_full = _RPS // _C
        rem = _RPS - q_full * _C
        GB, SB = (g0, g1), (sc0, sc1)
        GS, SS = (gs0, gs1), (ss0, ss1)
        ES = (es0, es1)

        def issue_gather(eb, i, par):
            pltpu.async_copy(tab_hbm.at[src_v.at[eb, i]], GB[par], GS[par])

        def wait_gather(par):
            pltpu.make_async_copy(tab_hbm.at[src_v.at[0, 0]],
                                  GB[par], GS[par]).wait()

        def issue_scatter(eb, i, par):
            pltpu.async_copy(SB[par], acc.at[dst_v.at[eb, i]], SS[par],
                             add=True)

        def wait_scatter(par):
            pltpu.make_async_copy(SB[par], acc.at[pl.ds(0, _C)],
                                  SS[par]).wait()

        def scale(eb, i, par):
            gbuf, sbuf = GB[par], SB[par]

            def rowblk(rb, _):
                r0 = rb * 16
                wvec = w_v[eb, i, pl.ds(r0, 16)]
                for t in range(16):
                    wv = wvec[t]
                    for k in range(F // 16):
                        sl = pl.ds(k * 16, 16)
                        sbuf[r0 + t, sl] = gbuf[r0 + t, sl] * wv
                return 0
            lax.fori_loop(0, _C // 16, rowblk, 0)

        def load_block(p, bi, eb, sync):
            trips = ((src_hbm.at[p, cid, sid, pl.ds(bi * _BLK, _BLK)],
                      src_v.at[eb]),
                     (dst_hbm.at[cid, sid, pl.ds(bi * _BLK, _BLK)],
                      dst_v.at[eb]),
                     (w_hbm.at[cid, sid, pl.ds(bi * _BLK, _BLK)],
                      w_v.at[eb]))
            if sync:
                for src, dst in trips:
                    pltpu.sync_copy(src, dst)
            else:
                for src, dst in trips:
                    pltpu.async_copy(src, dst, ES[eb])

        def wait_block(eb):
            pltpu.make_async_copy(src_hbm.at[0, cid, sid, pl.ds(0, _BLK)],
                                  src_v.at[eb], ES[eb]).wait()
            pltpu.make_async_copy(dst_hbm.at[cid, sid, pl.ds(0, _BLK)],
                                  dst_v.at[eb], ES[eb]).wait()
            pltpu.make_async_copy(w_hbm.at[cid, sid, pl.ds(0, _BLK)],
                                  w_v.at[eb], ES[eb]).wait()

        def steady(eb, i, par):
            wait_gather(par)
            wait_scatter(par)
            scale(eb, i, par)
            issue_scatter(eb, i, par)

        def zero_acc():
            def zr(r, _):
                for k in range(F // 16):
                    sc0[r, pl.ds(k * 16, 16)] = jnp.zeros((16,), jnp.float32)
                return 0
            lax.fori_loop(0, _C, zr, 0)

            @pl.when(sid < _WTILES)
            def _():
                for q in range(q_full):
                    pltpu.sync_copy(sc0, acc.at[pl.ds(base + q * _C, _C)])
                pltpu.sync_copy(sc0.at[pl.ds(0, rem)],
                                acc.at[pl.ds(base + q_full * _C, rem)])
            plsc.subcore_barrier()

        def writeout(p):
            plsc.subcore_barrier()

            @pl.when(sid < _WTILES)
            def _():
                pltpu.sync_copy(acc.at[pl.ds(base, _RPS)],
                                out_hbm.at[p, cid, pl.ds(base, _RPS)])
            plsc.subcore_barrier()

        def run_pass_blocked(p):
            def blk(bi, _):
                load_block(p, bi, 0, sync=True)
                issue_gather(0, 0, 0)
                issue_gather(0, 1, 1)
                for par in (0, 1):
                    wait_gather(par)
                    scale(0, par, par)
                    issue_scatter(0, par, par)
                    issue_gather(0, par + 2, par)

                def pair(kk, _):
                    for par in (0, 1):
                        i = kk * 2 + par
                        steady(0, i, par)
                        issue_gather(0, i + 2, par)
                    return 0
                lax.fori_loop(1, _BLK // 2 - 1, pair, 0)

                for par in (0, 1):
                    steady(0, _BLK - 2 + par, par)
                wait_scatter(0)
                wait_scatter(1)
                return 0
            lax.fori_loop(0, _NB, blk, 0)

        def run_pass_pipelined(p):
            load_block(p, 0, 0, sync=True)
            issue_gather(0, 0, 0)
            issue_gather(0, 1, 1)

            def block_body(b, eb, first, last):
                for par in (0, 1):
                    wait_gather(par)
                    if not first:
                        wait_scatter(par)
                    scale(eb, par, par)
                    issue_scatter(eb, par, par)
                    issue_gather(eb, par + 2, par)
                if not last:
                    load_block(p, b + 1, 1 - eb, sync=False)

                def pair(kk, _):
                    for par in (0, 1):
                        i = kk * 2 + par
                        steady(eb, i, par)
                        issue_gather(eb, i + 2, par)
                    return 0
                lax.fori_loop(1, _BLK // 2 - 1, pair, 0)

                for par in (0, 1):
                    steady(eb, _BLK - 2 + par, par)
                if not last:
                    wait_block(1 - eb)
                    issue_gather(1 - eb, 0, 0)
                    issue_gather(1 - eb, 1, 1)

            block_body(0, 0, True, False)

            def mid_blocks(bp, _):
                b = 1 + bp * 2
                block_body(b, 1, False, False)
                block_body(b + 1, 0, False, False)
                return 0
            lax.fori_loop(0, (_NB - 2) // 2, mid_blocks, 0)

            block_body(_NB - 1, 1, False, True)
            wait_scatter(0)
            wait_scatter(1)

        def one_pass(p, _):
            zero_acc()
            if pipelined:
                run_pass_pipelined(p)
            else:
                run_pass_blocked(p)
            writeout(p)
            return 0

        if NPASS == 1:
            one_pass(0, 0)
        else:
            lax.fori_loop(0, NPASS, one_pass, 0)

    return agg


@functools.partial(
    pl.kernel,
    out_type=jax.ShapeDtypeStruct((_NC, _NS, _N), jnp.float32),
    mesh=_mesh,
    scratch_types=[
        pltpu.VMEM((_BLK, _C), jnp.int32),
        pltpu.VMEM((_BLK, _C), jnp.float32),
        pltpu.VMEM((_N,), jnp.float32),
    ],
    compiler_params=pltpu.CompilerParams(use_tc_tiling_on_sc=False,
                                         needs_layout_passes=False),
)
def _deg(dst_hbm, w_hbm, out_hbm, dst_v, w_v, deg_v):
    cid = lax.axis_index("c")
    sid = lax.axis_index("s")

    def zr(r, _):
        deg_v[pl.ds(r * 16, 16)] = jnp.zeros((16,), jnp.float32)
        return 0
    lax.fori_loop(0, _N // 16, zr, 0)

    def blk(bi, _):
        b0 = bi * _BLK
        pltpu.sync_copy(dst_hbm.at[cid, sid, pl.ds(b0, _BLK)], dst_v)
        pltpu.sync_copy(w_hbm.at[cid, sid, pl.ds(b0, _BLK)], w_v)

        def chunk(i, _):
            for v in range(_C // 16):
                sl = pl.ds(v * 16, 16)
                plsc.addupdate_scatter(deg_v, [dst_v[i, sl]], w_v[i, sl])
            return 0
        lax.fori_loop(0, _BLK, chunk, 0)
        return 0
    lax.fori_loop(0, _NB, blk, 0)
    pltpu.sync_copy(deg_v, out_hbm.at[cid, sid])


_agg_32 = _make_agg(32, 1, False)
_agg_64_1 = _make_agg(64, 1, True)
_agg_64_2 = _make_agg(64, 2, True)


def _stage0_body(degp_ref, xt_ref, xf_ref, w1t_ref, w1f_ref,
                 dinv_ref, tab1_ref):
    for b, (x_ref, w_ref) in enumerate(((xt_ref, w1t_ref),
                                        (xf_ref, w1f_ref))):
        deg = jnp.sum(degp_ref[b], axis=0).reshape(_N, 1) + 1.0
        dinv = lax.rsqrt(jnp.maximum(deg, 1e-12))
        dinv_ref[b] = dinv
        h1 = jnp.dot(x_ref[...], w_ref[...],
                     preferred_element_type=jnp.float32)
        hs = h1 * dinv
        tab1_ref[0, pl.ds(b * _N, _N)] = hs[:, 0:64]
        tab1_ref[1, pl.ds(b * _N, _N)] = hs[:, 64:128]


def _stage0(degp, x_t, x_f, w1t, w1f):
    return pl.pallas_call(
        _stage0_body,
        compiler_params=pltpu.CompilerParams(
            vmem_limit_bytes=100 * 1024 * 1024),
        out_shape=(
            jax.ShapeDtypeStruct((2, _N, 1), jnp.float32),
            jax.ShapeDtypeStruct((2, 2 * _N, 64), jnp.float32),
        ),
    )(degp, x_t, x_f, w1t, w1f)


def _bn_relu(out, g, be):
    m = jnp.mean(out, axis=0, keepdims=True)
    v = jnp.mean((out - m) ** 2, axis=0, keepdims=True)
    return jax.nn.relu((out - m) * lax.rsqrt(v + 1e-5) * g + be)


def _mid_split_body(agg_ref, tab_ref, dinv_ref, pb_ref, pg_ref, pbe_ref,
                    w_ref, tabn_ref):
    o = dinv_ref[0] * (agg_ref[0, 0] + tab_ref[0]) + pb_ref[0, 0]
    y = _bn_relu(o, pg_ref[0, 0], pbe_ref[0, 0])
    part = jnp.dot(y, w_ref[0], preferred_element_type=jnp.float32)
    p2 = pl.program_id(1)

    @pl.when(p2 == 0)
    def _():
        tabn_ref[...] = part

    @pl.when(p2 == 1)
    def _():
        tabn_ref[...] = (tabn_ref[...] + part) * dinv_ref[0]


def _mid_body(agg_ref, tab_ref, dinv_ref, pb_ref, pg_ref, pbe_ref,
              w_ref, tabn_ref):
    dinv = dinv_ref[0]
    o = dinv * (agg_ref[0] + tab_ref[...]) + pb_ref[0]
    y = _bn_relu(o, pg_ref[0], pbe_ref[0])
    hn = jnp.dot(y, w_ref[0], preferred_element_type=jnp.float32)
    tabn_ref[...] = hn * dinv


def _mid(split, agg, tab, dinv, pb, pg, pbe, w2):
    fi, fo = w2.shape[1], w2.shape[2]
    if split:
        return pl.pallas_call(
            _mid_split_body,
            grid=(2, 2),
            in_specs=[
                pl.BlockSpec((1, 1, _N, 64), lambda b, q: (q, b, 0, 0)),
                pl.BlockSpec((1, _N, 64), lambda b, q: (q, b, 0)),
                pl.BlockSpec((1, _N, 1), lambda b, q: (b, 0, 0)),
                pl.BlockSpec((1, 1, 1, 64), lambda b, q: (b, q, 0, 0)),
                pl.BlockSpec((1, 1, 1, 64), lambda b, q: (b, q, 0, 0)),
                pl.BlockSpec((1, 1, 1, 64), lambda b, q: (b, q, 0, 0)),
                pl.BlockSpec((1, 64, fo), lambda b, q: (b, q, 0)),
            ],
            out_specs=pl.BlockSpec((_N, fo), lambda b, q: (b, 0)),
            compiler_params=pltpu.CompilerParams(
                vmem_limit_bytes=100 * 1024 * 1024),
            out_shape=jax.ShapeDtypeStruct((2 * _N, fo), jnp.float32),
        )(agg, tab, dinv, pb, pg, pbe, w2)
    return pl.pallas_call(
        _mid_body,
        grid=(2,),
        in_specs=[
            pl.BlockSpec((1, _N, fi), lambda b: (b, 0, 0)),
            pl.BlockSpec((_N, fi), lambda b: (b, 0)),
            pl.BlockSpec((1, _N, 1), lambda b: (b, 0, 0)),
            pl.BlockSpec((1, 1, fi), lambda b: (b, 0, 0)),
            pl.BlockSpec((1, 1, fi), lambda b: (b, 0, 0)),
            pl.BlockSpec((1, 1, fi), lambda b: (b, 0, 0)),
            pl.BlockSpec((1, fi, fo), lambda b: (b, 0, 0)),
        ],
        out_specs=pl.BlockSpec((_N, fo), lambda b: (b, 0)),
        compiler_params=pltpu.CompilerParams(
            vmem_limit_bytes=100 * 1024 * 1024),
        out_shape=jax.ShapeDtypeStruct((2 * _N, fo), jnp.float32),
    )(agg, tab, dinv, pb, pg, pbe, w2)


def _final_body(agg_ref, tab_ref, dinv_ref, pb_ref, pg_ref, pbe_ref,
                batch_ref, wp1_ref, bp1_ref, wp2_ref, bp2_ref,
                pool_ref, z_ref, x3_ref):
    o = (dinv_ref[0] * (agg_ref[0] + tab_ref[...]) + pb_ref[0])
    x3 = _bn_relu(o, pg_ref[0], pbe_ref[0])
    x3_ref[0] = x3
    gid = lax.broadcasted_iota(jnp.int32, (_N, _G), 1)
    onehot = (batch_ref[0] == gid).astype(jnp.float32)
    s = lax.dot_general(onehot, x3, (((0,), (0,)), ((), ())),
                        preferred_element_type=jnp.float32)
    c = jnp.sum(onehot, axis=0)
    pool = s / jnp.maximum(c, 1.0)[:, None]
    pool_ref[0] = pool
    z1 = jax.nn.relu(
        jnp.dot(pool, wp1_ref[0], preferred_element_type=jnp.float32)
        + bp1_ref[0])
    z_ref[0] = (jnp.dot(z1, wp2_ref[0],
                        preferred_element_type=jnp.float32)
                + bp2_ref[0])


def _final(agg, tab, dinv, pb, pg, pbe, batch2, wp1, bp1, wp2, bp2):
    pool, z, x3 = pl.pallas_call(
        _final_body,
        grid=(2,),
        in_specs=[
            pl.BlockSpec((1, _N, 32), lambda b: (b, 0, 0)),
            pl.BlockSpec((_N, 32), lambda b: (b, 0)),
            pl.BlockSpec((1, _N, 1), lambda b: (b, 0, 0)),
            pl.BlockSpec((1, 1, 32), lambda b: (b, 0, 0)),
            pl.BlockSpec((1, 1, 32), lambda b: (b, 0, 0)),
            pl.BlockSpec((1, 1, 32), lambda b: (b, 0, 0)),
            pl.BlockSpec((1, _N, 1), lambda b: (b, 0, 0)),
            pl.BlockSpec((1, 32, 64), lambda b: (b, 0, 0)),
            pl.BlockSpec((1, 1, 64), lambda b: (b, 0, 0)),
            pl.BlockSpec((1, 64, 32), lambda b: (b, 0, 0)),
            pl.BlockSpec((1, 1, 32), lambda b: (b, 0, 0)),
        ],
        out_specs=(
            pl.BlockSpec((1, _G, 32), lambda b: (b, 0, 0)),
            pl.BlockSpec((1, _G, 32), lambda b: (b, 0, 0)),
            pl.BlockSpec((1, _N, 32), lambda b: (b, 0, 0)),
        ),
        compiler_params=pltpu.CompilerParams(
            vmem_limit_bytes=100 * 1024 * 1024),
        out_shape=(
            jax.ShapeDtypeStruct((2, _G, 32), jnp.float32),
            jax.ShapeDtypeStruct((2, _G, 32), jnp.float32),
            jax.ShapeDtypeStruct((2, _N, 32), jnp.float32),
        ),
    )(agg, tab, dinv, pb, pg, pbe, batch2, wp1, bp1, wp2, bp2)
    return pool[0], z[0], pool[1], z[1], x3[0], x3[1]


def _pstack(pt, pf):
    return jnp.stack([pt.reshape(1, -1), pf.reshape(1, -1)])


def _pstack4(pt, pf):
    return jnp.stack([pt.reshape(2, 1, 64), pf.reshape(2, 1, 64)])


def kernel(x_t, edge_index_t, edge_attr_t, batch_t,
           x_f, edge_index_f, edge_attr_f, batch_f, params):
    p = params

    def eshape(a):
        return jnp.pad(a, (0, _EPAD - _E)).reshape(_NS, _PT, _C)

    src_all = jnp.stack([eshape(edge_index_t[0]),
                         eshape(edge_index_f[0]) + _N])
    dst_all = jnp.stack([eshape(edge_index_t[1]), eshape(edge_index_f[1])])
    w_all = jnp.stack([eshape(edge_attr_t), eshape(edge_attr_f)])
    src_1 = src_all[None]
    src_2 = jnp.stack([src_all, src_all + 2 * _N])

    degp = _deg(dst_all, w_all)

    dinv, tab1 = _stage0(degp, x_t, x_f, p['W1t'], p['W1f'])

    agg1 = _agg_64_2(tab1.reshape(4 * _N, 64), src_2, dst_all, w_all)
    tab2 = _mid(True, agg1, tab1, dinv, _pstack4(p['b1t'], p['b1f']),
                _pstack4(p['g_bn1t'], p['g_bn1f']),
                _pstack4(p['be_bn1t'], p['be_bn1f']),
                jnp.stack([p['W2t'], p['W2f']]))

    agg2 = _agg_64_1(tab2, src_1, dst_all, w_all)[0]
    tab3 = _mid(False, agg2, tab2, dinv, _pstack(p['b2t'], p['b2f']),
                _pstack(p['g_bn2t'], p['g_bn2f']),
                _pstack(p['be_bn2t'], p['be_bn2f']),
                jnp.stack([p['W3t'], p['W3f']]))

    agg3 = _agg_32(tab3, src_1, dst_all, w_all)[0]
    batch2 = jnp.stack([batch_t.reshape(_N, 1), batch_f.reshape(_N, 1)])
    h_time, z_time, h_freq, z_freq, xt, xf = _final(
        agg3, tab3, dinv, _pstack(p['b3t'], p['b3f']),
        _pstack(p['g_bn3t'], p['g_bn3f']), _pstack(p['be_bn3t'], p['be_bn3f']),
        batch2, jnp.stack([p['Wp1t'], p['Wp1f']]),
        _pstack(p['bp1t'], p['bp1f']), jnp.stack([p['Wp2t'], p['Wp2f']]),
        _pstack(p['bp2t'], p['bp2f']))

    return (h_time, z_time, h_freq, z_freq, xt, xf)
